# baseline (speedup 1.0000x reference)
import functools

import jax
import jax.numpy as jnp
from jax import lax
from jax.experimental import pallas as pl
from jax.experimental.pallas import tpu as pltpu
from jax.experimental.pallas import tpu_sc as plsc

N = 10000
NP = 10240
D = 128
E = 320000
NC, NS = 2, 16
NW = NC * NS
K = 128
NBLK = 80
NSLOT = 4
EP = NW * NBLK * K
RPT = NP // NS

_MESH = plsc.VectorSubcoreMesh(core_axis_name="c", subcore_axis_name="s")


def _deg_body(rows_hbm, cols_hbm, ones_hbm, zeros_hbm, out_hbm,
              dr_sh, dc_sh, rows_v, cols_v, ones_v,
              sem0, sem1, sem2, sem3, asem0, asem1, asem2, asem3):
    c = lax.axis_index("c")
    s = lax.axis_index("s")
    wid = s * NC + c
    sl = pl.ds(s * RPT, RPT)
    isems = (sem0, sem1, sem2, sem3)
    asems = (asem0, asem1, asem2, asem3)

    def _idx_load(j, slot):
        pltpu.async_copy(rows_hbm.at[wid, j], rows_v.at[slot], isems[slot])
        pltpu.async_copy(cols_hbm.at[wid, j], cols_v.at[slot], isems[slot])

    def _idx_wait(j, slot):
        pltpu.make_async_copy(rows_hbm.at[wid, j], rows_v.at[slot], isems[slot]).wait()
        pltpu.make_async_copy(cols_hbm.at[wid, j], cols_v.at[slot], isems[slot]).wait()

    def _fire(slot):
        pltpu.async_copy(ones_v, dr_sh.at[rows_v.at[slot]], asems[slot], add=True)
        pltpu.async_copy(ones_v, dc_sh.at[cols_v.at[slot]], asems[slot], add=True)

    def _drain(slot):
        pltpu.make_async_copy(ones_v, dr_sh.at[rows_v.at[slot]], asems[slot]).wait()
        pltpu.make_async_copy(ones_v, dc_sh.at[cols_v.at[slot]], asems[slot]).wait()

    for t in range(4):
        _idx_load(t, t)
    pltpu.async_copy(ones_hbm, ones_v, asem0)
    pltpu.async_copy(zeros_hbm, dr_sh.at[sl], asem0)
    pltpu.async_copy(zeros_hbm, dc_sh.at[sl], asem0)
    pltpu.make_async_copy(ones_hbm, ones_v, asem0).wait()
    pltpu.make_async_copy(zeros_hbm, dr_sh.at[sl], asem0).wait()
    pltpu.make_async_copy(zeros_hbm, dc_sh.at[sl], asem0).wait()
    plsc.subcore_barrier()

    _idx_wait(0, 0)
    _fire(0)
    _idx_wait(1, 1)
    _fire(1)

    @pl.loop(0, (NBLK - 4) // 4)
    def _(i):
        j0 = 2 + i * 4
        for t in range(4):
            cur = (2 + t) % 4
            _drain(t)
            _idx_load(j0 + t + 2, t)
            _idx_wait(j0 + t, cur)
            _fire(cur)

    _drain(0)
    _drain(1)
    _idx_wait(NBLK - 2, 2)
    _fire(2)
    _idx_wait(NBLK - 1, 3)
    _fire(3)
    _drain(2)
    _drain(3)
    plsc.subcore_barrier()
    pltpu.sync_copy(dr_sh.at[sl], out_hbm.at[c, 0, sl])
    pltpu.sync_copy(dc_sh.at[sl], out_hbm.at[c, 1, sl])


def _sc_degrees(rows, cols):
    ones = jnp.ones((K, 16), jnp.float32)
    zeros = jnp.zeros((RPT, 16), jnp.float32)
    f = pl.kernel(
        _deg_body,
        out_type=jax.ShapeDtypeStruct((NC, 2, NP, 16), jnp.float32),
        mesh=_MESH,
        scratch_types=[
            pltpu.VMEM_SHARED((NP, 16), jnp.float32),
            pltpu.VMEM_SHARED((NP, 16), jnp.float32),
            pltpu.VMEM((NSLOT, K), jnp.int32),
            pltpu.VMEM((NSLOT, K), jnp.int32),
            pltpu.VMEM((K, 16), jnp.float32),
        ] + [pltpu.SemaphoreType.DMA] * 8,
    )
    o = f(rows, cols, ones, zeros)
    return (o[0, 0, :N, 0] + o[1, 0, :N, 0],
            o[0, 1, :N, 0] + o[1, 1, :N, 0])


def _spmm_body(g_hbm, rows_hbm, cols_hbm, out_hbm, acc_sh, rows_v, cols_v,
               buf0, buf1, isem0, isem1, isem2, isem3, gsem0, gsem1,
               ssem0, ssem1):
    c = lax.axis_index("c")
    s = lax.axis_index("s")
    wid = s * NC + c
    sl = pl.ds(s * RPT, RPT)
    isems = (isem0, isem1, isem2, isem3)
    bufs = (buf0, buf1)
    gsems = (gsem0, gsem1)
    ssems = (ssem0, ssem1)

    def _idx_load(j, slot):
        pltpu.async_copy(rows_hbm.at[wid, j], rows_v.at[slot], isems[slot])
        pltpu.async_copy(cols_hbm.at[wid, j], cols_v.at[slot], isems[slot])

    def _idx_wait(j, slot):
        pltpu.make_async_copy(rows_hbm.at[wid, j], rows_v.at[slot], isems[slot]).wait()
        pltpu.make_async_copy(cols_hbm.at[wid, j], cols_v.at[slot], isems[slot]).wait()

    def _gather(slot, b):
        pltpu.async_copy(g_hbm.at[cols_v.at[slot]], bufs[b], gsems[b])

    def _gather_wait(slot, b):
        pltpu.make_async_copy(g_hbm.at[cols_v.at[slot]], bufs[b], gsems[b]).wait()

    def _scatter(slot, b):
        pltpu.async_copy(bufs[b], acc_sh.at[rows_v.at[slot]], ssems[b], add=True)

    def _scatter_wait(slot, b):
        pltpu.make_async_copy(bufs[b], acc_sh.at[rows_v.at[slot]], ssems[b]).wait()

    for t in range(4):
        _idx_load(t, t)
    pltpu.async_copy(g_hbm.at[sl], acc_sh.at[sl], gsem0)
    pltpu.make_async_copy(g_hbm.at[sl], acc_sh.at[sl], gsem0).wait()
    plsc.subcore_barrier()

    @pl.loop(0, NBLK // 4)
    def _(i):
        j0 = i * 4
        for t in range(4):
            _idx_wait(j0 + t, t)
            _gather(t, 0)
            _gather_wait(t, 0)
            _scatter(t, 0)
            _scatter_wait(t, 0)
            _idx_load((j0 + t + 4) % NBLK, t)

    for t in range(4):
        _idx_wait(t, t)
    plsc.subcore_barrier()
    pltpu.sync_copy(acc_sh.at[sl], out_hbm.at[c, sl])


def _sc_spmm(f, g, rows, cols):
    gp = jnp.zeros((NP, D), jnp.float32).at[:N].set(g)
    o = f(gp, rows, cols)
    return o[0, :N] + o[1, :N]


def _make_spmm():
    return pl.kernel(
        _spmm_body,
        out_type=jax.ShapeDtypeStruct((NC, NP, D), jnp.float32),
        mesh=_MESH,
        scratch_types=[
            pltpu.VMEM_SHARED((NP, D), jnp.float32),
            pltpu.VMEM((NSLOT, K), jnp.int32),
            pltpu.VMEM((NSLOT, K), jnp.int32),
            pltpu.VMEM((K, D), jnp.float32),
            pltpu.VMEM((K, D), jnp.float32),
        ] + [pltpu.SemaphoreType.DMA] * 8,
    )


ROW_BLK = 1000


def _mlp_body(x_ref, h1_ref, h2_ref, W0_ref, b0_ref, W1_ref, b1_ref,
              W2_ref, b2_ref, Wr_ref, br_ref, o_ref):
    n0 = jnp.maximum(jnp.dot(x_ref[...], W0_ref[...],
                             preferred_element_type=jnp.float32) + b0_ref[...], 0.0)
    n1 = jnp.maximum(jnp.dot(h1_ref[...], W1_ref[...],
                             preferred_element_type=jnp.float32) + b1_ref[...], 0.0)
    n2 = jnp.maximum(jnp.dot(h2_ref[...], W2_ref[...],
                             preferred_element_type=jnp.float32) + b2_ref[...], 0.0)
    agg = (n0 + n1 + n2) * (1.0 / 3.0)
    o_ref[...] = jnp.maximum(jnp.dot(agg, Wr_ref[...],
                                     preferred_element_type=jnp.float32) + br_ref[...], 0.0)


def _mlp_stage(x, h1, h2, W0, b0, W1, b1, W2, b2, Wr, br):
    row_spec = pl.BlockSpec((ROW_BLK, D), lambda i: (i, 0))
    w_spec = pl.BlockSpec((D, D), lambda i: (0, 0))
    b_spec = pl.BlockSpec((1, D), lambda i: (0, 0))
    return pl.pallas_call(
        _mlp_body,
        grid=(N // ROW_BLK,),
        in_specs=[row_spec, row_spec, row_spec,
                  w_spec, b_spec, w_spec, b_spec, w_spec, b_spec,
                  w_spec, b_spec],
        out_specs=row_spec,
        out_shape=jax.ShapeDtypeStruct((N, D), jnp.float32),
    )(x, h1, h2, W0, b0.reshape(1, D), W1, b1.reshape(1, D),
      W2, b2.reshape(1, D), Wr, br.reshape(1, D))


def kernel(x, edge_index, W0, b0, W1, b1, W2, b2, Wr, br):
    rows = edge_index[0].astype(jnp.int32)
    cols = edge_index[1].astype(jnp.int32)
    pad = jnp.full((EP - E,), NP - 1, jnp.int32)
    rows_p = jnp.concatenate([rows, pad]).reshape(NW, NBLK, K)
    cols_p = jnp.concatenate([cols, pad]).reshape(NW, NBLK, K)

    deg_r, deg_c = _sc_degrees(rows_p, cols_p)
    inv_r = jax.lax.rsqrt(deg_r + 1.0)[:, None]
    inv_c = jax.lax.rsqrt(deg_c + 1.0)[:, None]

    spmm = _make_spmm()
    g0 = x * inv_c
    s0 = _sc_spmm(spmm, g0, rows_p, cols_p)
    h1 = (s0 - g0) * inv_r
    g1 = h1 * inv_c
    s1 = _sc_spmm(spmm, g1, rows_p, cols_p)
    h2 = (s1 - g1) * inv_r

    return _mlp_stage(x, h1, h2, W0, b0, W1, b1, W2, b2, Wr, br)


# speedup vs baseline: 1.0003x; 1.0003x over previous
import functools

import jax
import jax.numpy as jnp
from jax import lax
from jax.experimental import pallas as pl
from jax.experimental.pallas import tpu as pltpu
from jax.experimental.pallas import tpu_sc as plsc

N = 10000
NP = 10240
D = 128
E = 320000
NC, NS = 2, 16
NW = NC * NS
K = 128
NBLK = 80
NSLOT = 4
EP = NW * NBLK * K
RPT = NP // NS

_MESH = plsc.VectorSubcoreMesh(core_axis_name="c", subcore_axis_name="s")


def _deg_body(rows_hbm, cols_hbm, ones_hbm, zeros_hbm, out_hbm,
              dr_sh, dc_sh, rows_v, cols_v, ones_v,
              sem0, sem1, sem2, sem3, asem0, asem1, asem2, asem3):
    c = lax.axis_index("c")
    s = lax.axis_index("s")
    wid = s * NC + c
    sl = pl.ds(s * RPT, RPT)
    isems = (sem0, sem1, sem2, sem3)
    asems = (asem0, asem1, asem2, asem3)

    def _idx_load(j, slot):
        pltpu.async_copy(rows_hbm.at[wid, j], rows_v.at[slot], isems[slot])
        pltpu.async_copy(cols_hbm.at[wid, j], cols_v.at[slot], isems[slot])

    def _idx_wait(j, slot):
        pltpu.make_async_copy(rows_hbm.at[wid, j], rows_v.at[slot], isems[slot]).wait()
        pltpu.make_async_copy(cols_hbm.at[wid, j], cols_v.at[slot], isems[slot]).wait()

    def _fire(slot):
        pltpu.async_copy(ones_v, dr_sh.at[rows_v.at[slot]], asems[slot], add=True)
        pltpu.async_copy(ones_v, dc_sh.at[cols_v.at[slot]], asems[slot], add=True)

    def _drain(slot):
        pltpu.make_async_copy(ones_v, dr_sh.at[rows_v.at[slot]], asems[slot]).wait()
        pltpu.make_async_copy(ones_v, dc_sh.at[cols_v.at[slot]], asems[slot]).wait()

    for t in range(4):
        _idx_load(t, t)
    pltpu.async_copy(ones_hbm, ones_v, asem0)
    pltpu.async_copy(zeros_hbm, dr_sh.at[sl], asem0)
    pltpu.async_copy(zeros_hbm, dc_sh.at[sl], asem0)
    pltpu.make_async_copy(ones_hbm, ones_v, asem0).wait()
    pltpu.make_async_copy(zeros_hbm, dr_sh.at[sl], asem0).wait()
    pltpu.make_async_copy(zeros_hbm, dc_sh.at[sl], asem0).wait()
    plsc.subcore_barrier()

    _idx_wait(0, 0)
    _fire(0)
    _idx_wait(1, 1)
    _fire(1)

    @pl.loop(0, (NBLK - 4) // 4)
    def _(i):
        j0 = 2 + i * 4
        for t in range(4):
            cur = (2 + t) % 4
            _drain(t)
            _idx_load(j0 + t + 2, t)
            _idx_wait(j0 + t, cur)
            _fire(cur)

    _drain(0)
    _drain(1)
    _idx_wait(NBLK - 2, 2)
    _fire(2)
    _idx_wait(NBLK - 1, 3)
    _fire(3)
    _drain(2)
    _drain(3)
    plsc.subcore_barrier()
    pltpu.sync_copy(dr_sh.at[sl], out_hbm.at[c, 0, sl])
    pltpu.sync_copy(dc_sh.at[sl], out_hbm.at[c, 1, sl])


def _sc_degrees(rows, cols):
    ones = jnp.ones((K, 16), jnp.float32)
    zeros = jnp.zeros((RPT, 16), jnp.float32)
    f = pl.kernel(
        _deg_body,
        out_type=jax.ShapeDtypeStruct((NC, 2, NP, 16), jnp.float32),
        mesh=_MESH,
        scratch_types=[
            pltpu.VMEM_SHARED((NP, 16), jnp.float32),
            pltpu.VMEM_SHARED((NP, 16), jnp.float32),
            pltpu.VMEM((NSLOT, K), jnp.int32),
            pltpu.VMEM((NSLOT, K), jnp.int32),
            pltpu.VMEM((K, 16), jnp.float32),
        ] + [pltpu.SemaphoreType.DMA] * 8,
    )
    o = f(rows, cols, ones, zeros)
    return (o[0, 0, :N, 0] + o[1, 0, :N, 0],
            o[0, 1, :N, 0] + o[1, 1, :N, 0])


def _spmm_body(g_hbm, rows_hbm, cols_hbm, out_hbm, acc_sh,
               r0, r1, r2, r3, c0, c1, c2, c3, buf,
               isem0, isem1, isem2, isem3, gsem, ssem):
    c = lax.axis_index("c")
    s = lax.axis_index("s")
    wid = s * NC + c
    sl = pl.ds(s * RPT, RPT)
    isems = (isem0, isem1, isem2, isem3)
    rring = (r0, r1, r2, r3)
    cring = (c0, c1, c2, c3)
    base = wid * (NBLK * K)

    def _idx_load(j, t):
        pltpu.async_copy(rows_hbm.at[pl.ds(base + j * K, K)], rring[t], isems[t])
        pltpu.async_copy(cols_hbm.at[pl.ds(base + j * K, K)], cring[t], isems[t])

    def _idx_wait(j, t):
        pltpu.make_async_copy(rows_hbm.at[pl.ds(base + j * K, K)], rring[t], isems[t]).wait()
        pltpu.make_async_copy(cols_hbm.at[pl.ds(base + j * K, K)], cring[t], isems[t]).wait()

    for t in range(4):
        _idx_load(t, t)
    pltpu.async_copy(g_hbm.at[sl], acc_sh.at[sl], gsem)
    pltpu.make_async_copy(g_hbm.at[sl], acc_sh.at[sl], gsem).wait()
    plsc.subcore_barrier()

    @pl.loop(0, NBLK // 4)
    def _(i):
        j0 = i * 4
        for t in range(4):
            _idx_wait(j0 + t, t)
            pltpu.sync_copy(g_hbm.at[cring[t]], buf)
            pltpu.sync_copy(buf, acc_sh.at[rring[t]], add=True)
            _idx_load((j0 + t + 4) % NBLK, t)

    for t in range(4):
        _idx_wait(t, t)
    plsc.subcore_barrier()
    pltpu.sync_copy(acc_sh.at[sl], out_hbm.at[c, sl])


def _sc_spmm(f, g, rows, cols):
    gp = jnp.zeros((NP, D), jnp.float32).at[:N].set(g)
    o = f(gp, rows, cols)
    return o[0, :N] + o[1, :N]


def _make_spmm():
    return pl.kernel(
        _spmm_body,
        out_type=jax.ShapeDtypeStruct((NC, NP, D), jnp.float32),
        mesh=_MESH,
        scratch_types=[
            pltpu.VMEM_SHARED((NP, D), jnp.float32),
        ] + [pltpu.VMEM((K,), jnp.int32)] * 8 + [
            pltpu.VMEM((K, D), jnp.float32),
        ] + [pltpu.SemaphoreType.DMA] * 6,
    )


ROW_BLK = 1000


def _mlp_body(x_ref, h1_ref, h2_ref, W0_ref, b0_ref, W1_ref, b1_ref,
              W2_ref, b2_ref, Wr_ref, br_ref, o_ref):
    n0 = jnp.maximum(jnp.dot(x_ref[...], W0_ref[...],
                             preferred_element_type=jnp.float32) + b0_ref[...], 0.0)
    n1 = jnp.maximum(jnp.dot(h1_ref[...], W1_ref[...],
                             preferred_element_type=jnp.float32) + b1_ref[...], 0.0)
    n2 = jnp.maximum(jnp.dot(h2_ref[...], W2_ref[...],
                             preferred_element_type=jnp.float32) + b2_ref[...], 0.0)
    agg = (n0 + n1 + n2) * (1.0 / 3.0)
    o_ref[...] = jnp.maximum(jnp.dot(agg, Wr_ref[...],
                                     preferred_element_type=jnp.float32) + br_ref[...], 0.0)


def _mlp_stage(x, h1, h2, W0, b0, W1, b1, W2, b2, Wr, br):
    row_spec = pl.BlockSpec((ROW_BLK, D), lambda i: (i, 0))
    w_spec = pl.BlockSpec((D, D), lambda i: (0, 0))
    b_spec = pl.BlockSpec((1, D), lambda i: (0, 0))
    return pl.pallas_call(
        _mlp_body,
        grid=(N // ROW_BLK,),
        in_specs=[row_spec, row_spec, row_spec,
                  w_spec, b_spec, w_spec, b_spec, w_spec, b_spec,
                  w_spec, b_spec],
        out_specs=row_spec,
        out_shape=jax.ShapeDtypeStruct((N, D), jnp.float32),
    )(x, h1, h2, W0, b0.reshape(1, D), W1, b1.reshape(1, D),
      W2, b2.reshape(1, D), Wr, br.reshape(1, D))


def kernel(x, edge_index, W0, b0, W1, b1, W2, b2, Wr, br):
    rows = edge_index[0].astype(jnp.int32)
    cols = edge_index[1].astype(jnp.int32)
    pad = jnp.full((EP - E,), NP - 1, jnp.int32)
    rows_p = jnp.concatenate([rows, pad]).reshape(NW, NBLK, K)
    cols_p = jnp.concatenate([cols, pad]).reshape(NW, NBLK, K)

    deg_r, deg_c = _sc_degrees(rows_p, cols_p)
    inv_r = jax.lax.rsqrt(deg_r + 1.0)[:, None]
    inv_c = jax.lax.rsqrt(deg_c + 1.0)[:, None]

    spmm = _make_spmm()
    rows_f = rows_p.reshape(EP)
    cols_f = cols_p.reshape(EP)
    g0 = x * inv_c
    s0 = _sc_spmm(spmm, g0, rows_f, cols_f)
    h1 = (s0 - g0) * inv_r
    g1 = h1 * inv_c
    s1 = _sc_spmm(spmm, g1, rows_f, cols_f)
    h2 = (s1 - g1) * inv_r

    return _mlp_stage(x, h1, h2, W0, b0, W1, b1, W2, b2, Wr, br)


# speedup vs baseline: 1.8283x; 1.8278x over previous
import functools

import jax
import jax.numpy as jnp
from jax import lax
from jax.experimental import pallas as pl
from jax.experimental.pallas import tpu as pltpu
from jax.experimental.pallas import tpu_sc as plsc

N = 10000
NP = 10240
D = 128
E = 320000
NC, NS = 2, 16
NW = NC * NS
E_W = E // NW
K = 80
NBLK = E_W // K
KD = 128
NBLKD = 80
NSLOT = 4
EP = NW * NBLKD * KD
RPT = NP // NS

_MESH = plsc.VectorSubcoreMesh(core_axis_name="c", subcore_axis_name="s")


def _deg_body(rows_hbm, cols_hbm, ones_hbm, zeros_hbm, out_hbm,
              dr_sh, dc_sh, rows_v, cols_v, ones_v,
              sem0, sem1, sem2, sem3, asem0, asem1, asem2, asem3):
    c = lax.axis_index("c")
    s = lax.axis_index("s")
    wid = s * NC + c
    sl = pl.ds(s * RPT, RPT)
    isems = (sem0, sem1, sem2, sem3)
    asems = (asem0, asem1, asem2, asem3)

    def _idx_load(j, slot):
        pltpu.async_copy(rows_hbm.at[wid, j], rows_v.at[slot], isems[slot])
        pltpu.async_copy(cols_hbm.at[wid, j], cols_v.at[slot], isems[slot])

    def _idx_wait(j, slot):
        pltpu.make_async_copy(rows_hbm.at[wid, j], rows_v.at[slot], isems[slot]).wait()
        pltpu.make_async_copy(cols_hbm.at[wid, j], cols_v.at[slot], isems[slot]).wait()

    def _fire(slot):
        pltpu.async_copy(ones_v, dr_sh.at[rows_v.at[slot]], asems[slot], add=True)
        pltpu.async_copy(ones_v, dc_sh.at[cols_v.at[slot]], asems[slot], add=True)

    def _drain(slot):
        pltpu.make_async_copy(ones_v, dr_sh.at[rows_v.at[slot]], asems[slot]).wait()
        pltpu.make_async_copy(ones_v, dc_sh.at[cols_v.at[slot]], asems[slot]).wait()

    for t in range(4):
        _idx_load(t, t)
    pltpu.async_copy(ones_hbm, ones_v, asem0)
    pltpu.async_copy(zeros_hbm, dr_sh.at[sl], asem0)
    pltpu.async_copy(zeros_hbm, dc_sh.at[sl], asem0)
    pltpu.make_async_copy(ones_hbm, ones_v, asem0).wait()
    pltpu.make_async_copy(zeros_hbm, dr_sh.at[sl], asem0).wait()
    pltpu.make_async_copy(zeros_hbm, dc_sh.at[sl], asem0).wait()
    plsc.subcore_barrier()

    _idx_wait(0, 0)
    _fire(0)
    _idx_wait(1, 1)
    _fire(1)

    @pl.loop(0, (NBLKD - 4) // 4)
    def _(i):
        j0 = 2 + i * 4
        for t in range(4):
            cur = (2 + t) % 4
            _drain(t)
            _idx_load(j0 + t + 2, t)
            _idx_wait(j0 + t, cur)
            _fire(cur)

    _drain(0)
    _drain(1)
    _idx_wait(NBLKD - 2, 2)
    _fire(2)
    _idx_wait(NBLKD - 1, 3)
    _fire(3)
    _drain(2)
    _drain(3)
    plsc.subcore_barrier()
    pltpu.sync_copy(dr_sh.at[sl], out_hbm.at[c, 0, sl])
    pltpu.sync_copy(dc_sh.at[sl], out_hbm.at[c, 1, sl])


def _sc_degrees(rows, cols):
    ones = jnp.ones((KD, 16), jnp.float32)
    zeros = jnp.zeros((RPT, 16), jnp.float32)
    f = pl.kernel(
        _deg_body,
        out_type=jax.ShapeDtypeStruct((NC, 2, NP, 16), jnp.float32),
        mesh=_MESH,
        scratch_types=[
            pltpu.VMEM_SHARED((NP, 16), jnp.float32),
            pltpu.VMEM_SHARED((NP, 16), jnp.float32),
            pltpu.VMEM((NSLOT, KD), jnp.int32),
            pltpu.VMEM((NSLOT, KD), jnp.int32),
            pltpu.VMEM((KD, 16), jnp.float32),
        ] + [pltpu.SemaphoreType.DMA] * 8,
    )
    o = f(rows, cols, ones, zeros)
    return (o[0, 0, :N, 0] + o[1, 0, :N, 0],
            o[0, 1, :N, 0] + o[1, 1, :N, 0])


def _spmm_body(g_hbm, rows_hbm, cols_hbm, out_hbm, acc_sh,
               ir0, ir1, ic0, ic1, buf0, buf1, ssem0, ssem1):
    c = lax.axis_index("c")
    s = lax.axis_index("s")
    wid = s * NC + c
    sl = pl.ds(s * RPT, RPT)
    irs = (ir0, ir1)
    ics = (ic0, ic1)
    bufs = (buf0, buf1)
    ssems = (ssem0, ssem1)
    base = wid * E_W

    pltpu.sync_copy(g_hbm.at[sl], acc_sh.at[sl])
    plsc.subcore_barrier()

    def _block(j, b, first):
        if not first:
            pltpu.make_async_copy(bufs[b], acc_sh.at[irs[b]], ssems[b]).wait()
        off = base + j * K
        pltpu.sync_copy(rows_hbm.at[pl.ds(off, K)], irs[b])
        pltpu.sync_copy(cols_hbm.at[pl.ds(off, K)], ics[b])
        pltpu.sync_copy(g_hbm.at[ics[b]], bufs[b])
        pltpu.async_copy(bufs[b], acc_sh.at[irs[b]], ssems[b], add=True)

    _block(0, 0, True)
    _block(1, 1, True)

    @pl.loop(1, (NBLK - 1) // 2)
    def _(i):
        j0 = i * 2
        _block(j0, 0, False)
        _block(j0 + 1, 1, False)

    _block(NBLK - 1, 0, False)
    pltpu.make_async_copy(bufs[0], acc_sh.at[irs[0]], ssems[0]).wait()
    pltpu.make_async_copy(bufs[1], acc_sh.at[irs[1]], ssems[1]).wait()
    plsc.subcore_barrier()
    pltpu.sync_copy(acc_sh.at[sl], out_hbm.at[c, sl])


def _sc_spmm(f, g, rows, cols):
    gp = jnp.zeros((NP, D), jnp.float32).at[:N].set(g)
    o = f(gp, rows, cols)
    return o[0, :N] + o[1, :N]


def _make_spmm():
    return pl.kernel(
        _spmm_body,
        out_type=jax.ShapeDtypeStruct((NC, NP, D), jnp.float32),
        mesh=_MESH,
        scratch_types=[
            pltpu.VMEM_SHARED((NP, D), jnp.float32),
            pltpu.VMEM((K,), jnp.int32),
            pltpu.VMEM((K,), jnp.int32),
            pltpu.VMEM((K,), jnp.int32),
            pltpu.VMEM((K,), jnp.int32),
            pltpu.VMEM((K, D), jnp.float32),
            pltpu.VMEM((K, D), jnp.float32),
            pltpu.SemaphoreType.DMA,
            pltpu.SemaphoreType.DMA,
        ],
    )


ROW_BLK = 1000


def _mlp_body(x_ref, h1_ref, h2_ref, W0_ref, b0_ref, W1_ref, b1_ref,
              W2_ref, b2_ref, Wr_ref, br_ref, o_ref):
    n0 = jnp.maximum(jnp.dot(x_ref[...], W0_ref[...],
                             preferred_element_type=jnp.float32) + b0_ref[...], 0.0)
    n1 = jnp.maximum(jnp.dot(h1_ref[...], W1_ref[...],
                             preferred_element_type=jnp.float32) + b1_ref[...], 0.0)
    n2 = jnp.maximum(jnp.dot(h2_ref[...], W2_ref[...],
                             preferred_element_type=jnp.float32) + b2_ref[...], 0.0)
    agg = (n0 + n1 + n2) * (1.0 / 3.0)
    o_ref[...] = jnp.maximum(jnp.dot(agg, Wr_ref[...],
                                     preferred_element_type=jnp.float32) + br_ref[...], 0.0)


def _mlp_stage(x, h1, h2, W0, b0, W1, b1, W2, b2, Wr, br):
    row_spec = pl.BlockSpec((ROW_BLK, D), lambda i: (i, 0))
    w_spec = pl.BlockSpec((D, D), lambda i: (0, 0))
    b_spec = pl.BlockSpec((1, D), lambda i: (0, 0))
    return pl.pallas_call(
        _mlp_body,
        grid=(N // ROW_BLK,),
        in_specs=[row_spec, row_spec, row_spec,
                  w_spec, b_spec, w_spec, b_spec, w_spec, b_spec,
                  w_spec, b_spec],
        out_specs=row_spec,
        out_shape=jax.ShapeDtypeStruct((N, D), jnp.float32),
    )(x, h1, h2, W0, b0.reshape(1, D), W1, b1.reshape(1, D),
      W2, b2.reshape(1, D), Wr, br.reshape(1, D))


def kernel(x, edge_index, W0, b0, W1, b1, W2, b2, Wr, br):
    rows = edge_index[0].astype(jnp.int32)
    cols = edge_index[1].astype(jnp.int32)
    pad = jnp.full((EP - E,), NP - 1, jnp.int32)
    rows_p = jnp.concatenate([rows, pad]).reshape(NW, NBLKD, KD)
    cols_p = jnp.concatenate([cols, pad]).reshape(NW, NBLKD, KD)

    deg_r, deg_c = _sc_degrees(rows_p, cols_p)
    inv_r = jax.lax.rsqrt(deg_r + 1.0)[:, None]
    inv_c = jax.lax.rsqrt(deg_c + 1.0)[:, None]

    spmm = _make_spmm()
    g0 = x * inv_c
    s0 = _sc_spmm(spmm, g0, rows, cols)
    h1 = (s0 - g0) * inv_r
    g1 = h1 * inv_c
    s1 = _sc_spmm(spmm, g1, rows, cols)
    h2 = (s1 - g1) * inv_r

    return _mlp_stage(x, h1, h2, W0, b0, W1, b1, W2, b2, Wr, br)
